# Initial kernel scaffold; baseline (speedup 1.0000x reference)
import functools

import jax
import jax.numpy as jnp
from jax import lax
from jax.experimental import pallas as pl
from jax.experimental.pallas import tpu as pltpu
from jax.experimental.pallas import tpu_sc as plsc

EMBED = 32
IDX_COLS = 128
STEP_ROWS = 8


def _make_gather(n_rows):
    info = plsc.get_sparse_core_info()
    nc, ns = info.num_cores, info.num_subcores
    nw = nc * ns
    rows_per_w = n_rows // nw
    steps = rows_per_w // STEP_ROWS
    chunk = STEP_ROWS * IDX_COLS

    mesh = plsc.VectorSubcoreMesh(core_axis_name="c", subcore_axis_name="s")

    @functools.partial(
        pl.kernel,
        mesh=mesh,
        out_type=jax.ShapeDtypeStruct((n_rows * IDX_COLS, EMBED), jnp.float32),
        scratch_types=[
            pltpu.VMEM((STEP_ROWS, IDX_COLS), jnp.int32),
            pltpu.VMEM((chunk, EMBED), jnp.float32),
            pltpu.SemaphoreType.DMA,
        ],
    )
    def k(table_hbm, idx_hbm, out_hbm, idx_v, rows_v, sem):
        wid = lax.axis_index("s") * nc + lax.axis_index("c")
        row_base = wid * rows_per_w

        def body(step, carry):
            r0 = row_base + step * STEP_ROWS
            pltpu.sync_copy(idx_hbm.at[pl.ds(r0, STEP_ROWS), :], idx_v)
            cps = []
            for j in range(STEP_ROWS):
                cps.append(
                    pltpu.async_copy(
                        table_hbm.at[idx_v.at[j]],
                        rows_v.at[pl.ds(j * IDX_COLS, IDX_COLS), :],
                        sem,
                    )
                )
            for cp in cps:
                cp.wait()
            pltpu.sync_copy(rows_v, out_hbm.at[pl.ds(r0 * IDX_COLS, chunk), :])
            return carry

        lax.fori_loop(0, steps, body, 0)

    return k


def kernel(location, table):
    b, h = location.shape
    idx = location.reshape(-1).astype(jnp.int32).reshape(-1, IDX_COLS)
    out = _make_gather(idx.shape[0])(table, idx)
    return out.reshape(b, h, EMBED)


# speedup vs baseline: 4.5560x; 4.5560x over previous
import functools

import jax
import jax.numpy as jnp
from jax import lax
from jax.experimental import pallas as pl
from jax.experimental.pallas import tpu as pltpu
from jax.experimental.pallas import tpu_sc as plsc

EMBED = 32
IDX_COLS = 128
STEP_ROWS = 8


def _make_gather(n_rows):
    info = plsc.get_sparse_core_info()
    nc, ns = info.num_cores, info.num_subcores
    nw = nc * ns
    rows_per_w = n_rows // nw
    steps = rows_per_w // STEP_ROWS
    chunk = STEP_ROWS * IDX_COLS

    mesh = plsc.VectorSubcoreMesh(core_axis_name="c", subcore_axis_name="s")

    @functools.partial(
        pl.kernel,
        mesh=mesh,
        compiler_params=pltpu.CompilerParams(use_tc_tiling_on_sc=False),
        out_type=jax.ShapeDtypeStruct((n_rows * IDX_COLS, EMBED), jnp.float32),
        scratch_types=[
            pltpu.VMEM((STEP_ROWS, IDX_COLS), jnp.int32),
            pltpu.VMEM((chunk, EMBED), jnp.float32),
            pltpu.SemaphoreType.DMA,
        ],
    )
    def k(table_hbm, idx_hbm, out_hbm, idx_v, rows_v, sem):
        wid = lax.axis_index("s") * nc + lax.axis_index("c")
        row_base = wid * rows_per_w

        def body(step, carry):
            r0 = row_base + step * STEP_ROWS
            pltpu.sync_copy(idx_hbm.at[pl.ds(r0, STEP_ROWS), :], idx_v)
            cps = []
            for j in range(STEP_ROWS):
                cps.append(
                    pltpu.async_copy(
                        table_hbm.at[idx_v.at[j]],
                        rows_v.at[pl.ds(j * IDX_COLS, IDX_COLS), :],
                        sem,
                    )
                )
            for cp in cps:
                cp.wait()
            pltpu.sync_copy(rows_v, out_hbm.at[pl.ds(r0 * IDX_COLS, chunk), :])
            return carry

        lax.fori_loop(0, steps, body, 0)

    return k


def kernel(location, table):
    b, h = location.shape
    idx = location.reshape(-1).astype(jnp.int32).reshape(-1, IDX_COLS)
    out = _make_gather(idx.shape[0])(table, idx)
    return out.reshape(b, h, EMBED)
